# baseline (speedup 1.0000x reference)
import functools

import jax
import jax.numpy as jnp
from jax import lax
from jax.experimental import pallas as pl
from jax.experimental.pallas import tpu as pltpu
from jax.experimental.pallas import tpu_sc as plsc

N_NODES = 10000
N_EDGES = 320000
D_FEAT = 128

NC = 2
NS = 16

SPLIT = 5120
ROWS0 = SPLIT
ROWS1 = N_NODES - SPLIT
ACC_ROWS = 5376
TRASH = 5248

E_PER_SUB = N_EDGES // NS
NR = 5
R_EDGES = E_PER_SUB // NR
BATCH = 128
CAPB = 33


def _sc_body(src_hbm, dst_hbm, x_hbm, out_hbm,
             sblk0, sblk1, dblk0, dblk1, csrc0, csrc1, cdst0, cdst1,
             rows0, rows1, zbuf,
             sem_g0, sem_g1, sem_es, sem_ed, acc):
    i32 = jnp.int32
    c = lax.axis_index("c").astype(i32)
    s = lax.axis_index("s").astype(i32)
    ebase = s * i32(E_PER_SUB)

    sblks = (sblk0, sblk1)
    dblks = (dblk0, dblk1)
    csrcs = (csrc0, csrc1)
    cdsts = (cdst0, cdst1)

    def stage_start(r):
        rb = r % 2
        off = ebase + i32(r * R_EDGES)
        pltpu.make_async_copy(src_hbm.at[pl.ds(off, R_EDGES)],
                              sblks[rb], sem_es).start()
        pltpu.make_async_copy(dst_hbm.at[pl.ds(off, R_EDGES)],
                              dblks[rb], sem_ed).start()

    def stage_wait(r):
        rb = r % 2
        off = ebase + i32(r * R_EDGES)
        pltpu.make_async_copy(src_hbm.at[pl.ds(off, R_EDGES)],
                              sblks[rb], sem_es).wait()
        pltpu.make_async_copy(dst_hbm.at[pl.ds(off, R_EDGES)],
                              dblks[rb], sem_ed).wait()

    stage_start(0)

    z16 = jnp.zeros((16,), jnp.float32)
    for r in range(16):
        for j in range(D_FEAT // 16):
            zbuf[r, pl.ds(j * 16, 16)] = z16

    zrows = ACC_ROWS // NS

    @pl.loop(i32(0), i32(zrows // 16))
    def _zero(g):
        pltpu.sync_copy(zbuf, acc.at[pl.ds(s * i32(zrows) + g * i32(16), 16)])

    plsc.subcore_barrier()

    lo = c * i32(SPLIT)
    hi = i32(SPLIT) + c * i32(ROWS1)
    lanes = lax.iota(jnp.int32, 16)
    pad_src = s * i32(16) + lanes
    pad_dst = i32(TRASH) + lanes

    bufs = (rows0, rows1)
    gsems = (sem_g0, sem_g1)

    for r in range(NR):
        sblk, dblk = sblks[r % 2], dblks[r % 2]
        csrc, cdst = csrcs[r % 2], cdsts[r % 2]
        stage_wait(r)
        if r + 1 < NR:
            stage_start(r + 1)

        def scan_step(k, carry, sblk=sblk, dblk=dblk, csrc=csrc, cdst=cdst):
            cnt, off = carry
            d16 = dblk[pl.ds(off, 16)]
            msk = (d16 >= lo) & (d16 < hi)
            s16 = sblk[pl.ds(off, 16)]
            mi = msk.astype(jnp.int32)
            pos = cnt + plsc.cumsum(mi) - i32(1)
            plsc.store_scatter(csrc, (pos,), s16, mask=msk)
            plsc.store_scatter(cdst, (pos >> 7, pos & i32(127)),
                               d16 - lo, mask=msk)
            return (cnt + jnp.sum(mi, dtype=jnp.int32), off + i32(16))

        cnt, _ = lax.fori_loop(0, R_EDGES // 16, scan_step,
                               (i32(0), i32(0)), unroll=4)

        for k in range(8):
            p = cnt + i32(16 * k) + lanes
            plsc.store_scatter(csrc, (p,), pad_src)
            plsc.store_scatter(cdst, (p >> 7, p & i32(127)), pad_dst)

        nb = (cnt + i32(127)) // i32(128)

        def gather_start(j, b, csrc=csrc):
            pltpu.make_async_copy(
                x_hbm.at[csrc.at[pl.ds(j * i32(BATCH), BATCH)]],
                bufs[b], gsems[b]).start()

        def gather_wait(j, b, csrc=csrc):
            pltpu.make_async_copy(
                x_hbm.at[csrc.at[pl.ds(j * i32(BATCH), BATCH)]],
                bufs[b], gsems[b]).wait()

        def accumulate(j, b, cdst=cdst):
            pltpu.sync_copy(bufs[b], acc.at[cdst.at[j]], add=True)

        @pl.when(nb > i32(0))
        def _prologue():
            gather_start(i32(0), 0)

        @pl.when(nb > i32(1))
        def _prologue2():
            gather_start(i32(1), 1)

        def gs_step(g, carry):
            for b in range(2):
                j = g * i32(2) + i32(b)

                @pl.when(j < nb)
                def _do(j=j, b=b):
                    gather_wait(j, b)
                    accumulate(j, b)

                    @pl.when(j + i32(2) < nb)
                    def _next():
                        gather_start(j + i32(2), b)
            return carry

        lax.fori_loop(i32(0), (nb + i32(1)) // i32(2), gs_step, i32(0))

    plsc.subcore_barrier()

    @pl.when(c == i32(0))
    def _out0():
        rr = ROWS0 // NS
        pltpu.sync_copy(acc.at[pl.ds(s * i32(rr), rr)],
                        out_hbm.at[pl.ds(s * i32(rr), rr)])

    @pl.when(c == i32(1))
    def _out1():
        rr = 304
        pltpu.sync_copy(acc.at[pl.ds(s * i32(rr), rr)],
                        out_hbm.at[pl.ds(i32(SPLIT) + s * i32(rr), rr)])

    @pl.when((c == i32(1)) & (s == i32(0)))
    def _out1_tail():
        pltpu.sync_copy(acc.at[pl.ds(i32(4864), 16)],
                        out_hbm.at[pl.ds(i32(9984), 16)])


@functools.partial(
    pl.kernel,
    out_type=jax.ShapeDtypeStruct((N_NODES, D_FEAT), jnp.float32),
    mesh=plsc.VectorSubcoreMesh(core_axis_name="c", subcore_axis_name="s",
                                num_cores=NC, num_subcores=NS),
    compiler_params=pltpu.CompilerParams(needs_layout_passes=False),
    scratch_types=[
        pltpu.VMEM((R_EDGES,), jnp.int32),
        pltpu.VMEM((R_EDGES,), jnp.int32),
        pltpu.VMEM((R_EDGES,), jnp.int32),
        pltpu.VMEM((R_EDGES,), jnp.int32),
        pltpu.VMEM((CAPB * BATCH,), jnp.int32),
        pltpu.VMEM((CAPB * BATCH,), jnp.int32),
        pltpu.VMEM((CAPB, BATCH), jnp.int32),
        pltpu.VMEM((CAPB, BATCH), jnp.int32),
        pltpu.VMEM((BATCH, D_FEAT), jnp.float32),
        pltpu.VMEM((BATCH, D_FEAT), jnp.float32),
        pltpu.VMEM((16, D_FEAT), jnp.float32),
        pltpu.SemaphoreType.DMA,
        pltpu.SemaphoreType.DMA,
        pltpu.SemaphoreType.DMA,
        pltpu.SemaphoreType.DMA,
        pltpu.VMEM_SHARED((ACC_ROWS, D_FEAT), jnp.float32),
    ],
)
def _sc_scatter(src_hbm, dst_hbm, x_hbm, out_hbm, *scratch):
    _sc_body(src_hbm, dst_hbm, x_hbm, out_hbm, *scratch)


def kernel(x, edge_index):
    ei = edge_index.astype(jnp.int32)
    return _sc_scatter(ei[0], ei[1], x)


# speedup vs baseline: 1.0702x; 1.0702x over previous
import functools

import jax
import jax.numpy as jnp
from jax import lax
from jax.experimental import pallas as pl
from jax.experimental.pallas import tpu as pltpu
from jax.experimental.pallas import tpu_sc as plsc

N_NODES = 10000
N_EDGES = 320000
D_FEAT = 128

NC = 2
NS = 16

SPLIT = 5120
ROWS0 = SPLIT
ROWS1 = N_NODES - SPLIT
ACC_ROWS = 5376
TRASH = 5248

E_PER_SUB = N_EDGES // NS
NR = 5
R_EDGES = E_PER_SUB // NR
BATCH = 128
CAPB = 33


def _sc_body(src_hbm, dst_hbm, x_hbm, out_hbm,
             sblk0, sblk1, dblk0, dblk1, csrc0, csrc1, cdst0, cdst1,
             rows0, rows1, zbuf,
             sem_g0, sem_g1, sem_es, sem_ed, acc):
    i32 = jnp.int32
    c = lax.axis_index("c").astype(i32)
    s = lax.axis_index("s").astype(i32)
    ebase = s * i32(E_PER_SUB)

    sblks = (sblk0, sblk1)
    dblks = (dblk0, dblk1)
    csrcs = (csrc0, csrc1)
    cdsts = (cdst0, cdst1)

    def stage_start(r):
        rb = r % 2
        off = ebase + i32(r * R_EDGES)
        pltpu.make_async_copy(src_hbm.at[pl.ds(off, R_EDGES)],
                              sblks[rb], sem_es).start()
        pltpu.make_async_copy(dst_hbm.at[pl.ds(off, R_EDGES)],
                              dblks[rb], sem_ed).start()

    def stage_wait(r):
        rb = r % 2
        off = ebase + i32(r * R_EDGES)
        pltpu.make_async_copy(src_hbm.at[pl.ds(off, R_EDGES)],
                              sblks[rb], sem_es).wait()
        pltpu.make_async_copy(dst_hbm.at[pl.ds(off, R_EDGES)],
                              dblks[rb], sem_ed).wait()

    stage_start(0)

    z16 = jnp.zeros((16,), jnp.float32)
    for r in range(16):
        for j in range(D_FEAT // 16):
            zbuf[r, pl.ds(j * 16, 16)] = z16

    zrows = ACC_ROWS // NS

    @pl.loop(i32(0), i32(zrows // 16))
    def _zero(g):
        pltpu.sync_copy(zbuf, acc.at[pl.ds(s * i32(zrows) + g * i32(16), 16)])

    plsc.subcore_barrier()

    lo = c * i32(SPLIT)
    hi = i32(SPLIT) + c * i32(ROWS1)
    lanes = lax.iota(jnp.int32, 16)
    pad_src = s * i32(16) + lanes
    pad_dst = i32(TRASH) + lanes

    bufs = (rows0, rows1)
    gsems = (sem_g0, sem_g1)

    def make_scan_step(sblk, dblk, csrc, cdst):
        def step(k, carry):
            cnt, off = carry
            offc = jnp.minimum(off, i32(R_EDGES - 16))
            d16 = dblk[pl.ds(offc, 16)]
            valid = off < i32(R_EDGES)
            msk = (d16 >= lo) & (d16 < hi) & valid
            s16 = sblk[pl.ds(offc, 16)]
            mi = msk.astype(jnp.int32)
            pos = cnt + plsc.cumsum(mi) - i32(1)
            plsc.store_scatter(csrc, (pos,), s16, mask=msk)
            plsc.store_scatter(cdst, (pos >> 7, pos & i32(127)),
                               d16 - lo, mask=msk)
            return (cnt + jnp.sum(mi, dtype=jnp.int32), off + i32(16))

        return step

    def write_pads(csrc, cdst, cnt):
        for k in range(8):
            p = cnt + i32(16 * k) + lanes
            plsc.store_scatter(csrc, (p,), pad_src)
            plsc.store_scatter(cdst, (p >> 7, p & i32(127)), pad_dst)

    sblks = (sblk0, sblk1)
    dblks = (dblk0, dblk1)
    csrcs = (csrc0, csrc1)
    cdsts = (cdst0, cdst1)

    stage_wait(0)
    stage_start(1)
    step0 = make_scan_step(sblk0, dblk0, csrc0, cdst0)
    cnt_cur, _ = lax.fori_loop(0, R_EDGES // 16, step0, (i32(0), i32(0)),
                               unroll=4)

    SCAN_S = 16

    for r in range(NR):
        csrc, cdst = csrcs[r % 2], cdsts[r % 2]
        write_pads(csrc, cdst, cnt_cur)
        nb = (cnt_cur + i32(127)) // i32(128)

        def gather_start(j, b, csrc=csrc):
            pltpu.make_async_copy(
                x_hbm.at[csrc.at[pl.ds(j * i32(BATCH), BATCH)]],
                bufs[b], gsems[b]).start()

        def gather_wait(j, b, csrc=csrc):
            pltpu.make_async_copy(
                x_hbm.at[csrc.at[pl.ds(j * i32(BATCH), BATCH)]],
                bufs[b], gsems[b]).wait()

        def accumulate(j, b, cdst=cdst):
            pltpu.sync_copy(bufs[b], acc.at[cdst.at[j]], add=True)

        @pl.when(nb > i32(0))
        def _prologue():
            gather_start(i32(0), 0)

        @pl.when(nb > i32(1))
        def _prologue2():
            gather_start(i32(1), 1)

        if r + 1 < NR:
            stage_wait(r + 1)
            if r + 2 < NR:
                stage_start(r + 2)
            nstep = make_scan_step(sblks[(r + 1) % 2], dblks[(r + 1) % 2],
                                   csrcs[(r + 1) % 2], cdsts[(r + 1) % 2])

            def gs_step(g, carry, nb=nb, gather_wait=gather_wait,
                        accumulate=accumulate, gather_start=gather_start,
                        nstep=nstep):
                for b in range(2):
                    j = g * i32(2) + i32(b)

                    @pl.when(j < nb)
                    def _do(j=j, b=b):
                        gather_wait(j, b)
                        accumulate(j, b)

                        @pl.when(j + i32(2) < nb)
                        def _next():
                            gather_start(j + i32(2), b)

                    carry = lax.fori_loop(0, SCAN_S, nstep, carry,
                                          unroll=2)
                return carry

            carry = lax.fori_loop(i32(0), (nb + i32(1)) // i32(2), gs_step,
                                  (i32(0), i32(0)))
            cnt_nxt, off_nxt = carry
            rem = (i32(R_EDGES) - off_nxt) // i32(16)
            cnt_cur, _ = lax.fori_loop(i32(0), rem, nstep,
                                       (cnt_nxt, off_nxt))
        else:
            def gs_step(g, carry, nb=nb, gather_wait=gather_wait,
                        accumulate=accumulate, gather_start=gather_start):
                for b in range(2):
                    j = g * i32(2) + i32(b)

                    @pl.when(j < nb)
                    def _do(j=j, b=b):
                        gather_wait(j, b)
                        accumulate(j, b)

                        @pl.when(j + i32(2) < nb)
                        def _next():
                            gather_start(j + i32(2), b)
                return carry

            lax.fori_loop(i32(0), (nb + i32(1)) // i32(2), gs_step, i32(0))

    plsc.subcore_barrier()

    @pl.when(c == i32(0))
    def _out0():
        rr = ROWS0 // NS
        pltpu.sync_copy(acc.at[pl.ds(s * i32(rr), rr)],
                        out_hbm.at[pl.ds(s * i32(rr), rr)])

    @pl.when(c == i32(1))
    def _out1():
        rr = 304
        pltpu.sync_copy(acc.at[pl.ds(s * i32(rr), rr)],
                        out_hbm.at[pl.ds(i32(SPLIT) + s * i32(rr), rr)])

    @pl.when((c == i32(1)) & (s == i32(0)))
    def _out1_tail():
        pltpu.sync_copy(acc.at[pl.ds(i32(4864), 16)],
                        out_hbm.at[pl.ds(i32(9984), 16)])


@functools.partial(
    pl.kernel,
    out_type=jax.ShapeDtypeStruct((N_NODES, D_FEAT), jnp.float32),
    mesh=plsc.VectorSubcoreMesh(core_axis_name="c", subcore_axis_name="s",
                                num_cores=NC, num_subcores=NS),
    compiler_params=pltpu.CompilerParams(needs_layout_passes=False),
    scratch_types=[
        pltpu.VMEM((R_EDGES,), jnp.int32),
        pltpu.VMEM((R_EDGES,), jnp.int32),
        pltpu.VMEM((R_EDGES,), jnp.int32),
        pltpu.VMEM((R_EDGES,), jnp.int32),
        pltpu.VMEM((CAPB * BATCH,), jnp.int32),
        pltpu.VMEM((CAPB * BATCH,), jnp.int32),
        pltpu.VMEM((CAPB, BATCH), jnp.int32),
        pltpu.VMEM((CAPB, BATCH), jnp.int32),
        pltpu.VMEM((BATCH, D_FEAT), jnp.float32),
        pltpu.VMEM((BATCH, D_FEAT), jnp.float32),
        pltpu.VMEM((16, D_FEAT), jnp.float32),
        pltpu.SemaphoreType.DMA,
        pltpu.SemaphoreType.DMA,
        pltpu.SemaphoreType.DMA,
        pltpu.SemaphoreType.DMA,
        pltpu.VMEM_SHARED((ACC_ROWS, D_FEAT), jnp.float32),
    ],
)
def _sc_scatter(src_hbm, dst_hbm, x_hbm, out_hbm, *scratch):
    _sc_body(src_hbm, dst_hbm, x_hbm, out_hbm, *scratch)


def kernel(x, edge_index):
    ei = edge_index.astype(jnp.int32)
    return _sc_scatter(ei[0], ei[1], x)
